# baseline (speedup 1.0000x reference)
import functools

import jax
import jax.numpy as jnp
from jax import lax
from jax.experimental import pallas as pl
from jax.experimental.pallas import tpu as pltpu
from jax.experimental.pallas import tpu_sc as plsc

B = 16384
D = 32
NC = 2
NS = 16
NW = NC * NS
BPW = B // NW
HALF = 8
NHALF = BPW // HALF
PAD = BPW + 16


def _gmf_body(users_hbm, items_hbm, ut_hbm, it_hbm, w_hbm, b_hbm, out_hbm,
              uidx_v, iidx_v, uw_v, iw_v, uc_v, w_v, b_v, out_v,
              sem_u, sem_i):
    wid = lax.axis_index("s") * NC + lax.axis_index("c")
    base = wid * BPW

    pltpu.sync_copy(users_hbm.at[pl.ds(base, BPW)], uidx_v.at[pl.ds(0, BPW)])
    pltpu.sync_copy(items_hbm.at[pl.ds(base, BPW)], iidx_v.at[pl.ds(0, BPW)])
    pltpu.sync_copy(w_hbm, w_v)
    pltpu.sync_copy(b_hbm, b_v.at[pl.ds(0, 1)])

    wv0 = w_v[0, pl.ds(0, 16)]
    wv1 = w_v[0, pl.ds(16, 16)]
    bias = b_v[pl.ds(0, 16)][0]
    lane16 = lax.iota(jnp.int32, 16)

    def fire(tab_hbm, win_v, ivec, sem):
        for j in range(HALF):
            c = ivec[j]
            c128 = pl.multiple_of((c >> 7) << 7, 128)
            pltpu.async_copy(
                tab_hbm.at[:, pl.ds(c128, 128)], win_v.at[j], sem)

    def drain(win_v, sem):
        for j in range(HALF):
            pltpu.make_async_copy(
                ut_hbm.at[:, pl.ds(0, 128)], win_v.at[j], sem).wait()

    def half_idx(idx_v, h):
        return idx_v[pl.ds(h * HALF, 16)]

    def extract(win_v, lv, dst_half):
        wq = lane16 & 7
        for d in range(D):
            dsplat = jnp.full((16,), d, jnp.int32)
            v = plsc.load_gather(win_v, [wq, dsplat, lv])
            uc_v[dst_half * D + d] = v

    fire(ut_hbm, uw_v, half_idx(uidx_v, 0), sem_u)

    def half_group(h, carry):
        iv_u = half_idx(uidx_v, h)
        iv_i = half_idx(iidx_v, h)
        fire(it_hbm, iw_v, iv_i, sem_i)
        drain(uw_v, sem_u)
        extract(uw_v, iv_u & 127, 0)
        fire(ut_hbm, uw_v, half_idx(uidx_v, h + 1), sem_u)
        drain(iw_v, sem_i)
        extract(iw_v, iv_i & 127, 1)
        acc = jnp.full((16,), bias, jnp.float32)
        for d in range(D):
            wd = wv0[d] if d < 16 else wv1[d - 16]
            acc = acc + (uc_v[d] * uc_v[D + d]) * wd
        out_v[pl.ds(h * HALF, 16)] = jnp.where(lane16 < HALF, acc, 0.0)
        return carry

    lax.fori_loop(0, NHALF - 1, half_group, 0)
    h_last = NHALF - 1
    iv_i = half_idx(iidx_v, h_last)
    fire(it_hbm, iw_v, iv_i, sem_i)
    drain(uw_v, sem_u)
    extract(uw_v, half_idx(uidx_v, h_last) & 127, 0)
    drain(iw_v, sem_i)
    extract(iw_v, iv_i & 127, 1)
    acc = jnp.full((16,), bias, jnp.float32)
    for d in range(D):
        wd = wv0[d] if d < 16 else wv1[d - 16]
        acc = acc + (uc_v[d] * uc_v[D + d]) * wd
    out_v[pl.ds(h_last * HALF, 16)] = jnp.where(lane16 < HALF, acc, 0.0)

    pltpu.sync_copy(out_v.at[pl.ds(0, BPW)], out_hbm.at[pl.ds(base, BPW)])


@jax.jit
def _gmf(users, items, user_table, item_table, W, b):
    mesh = plsc.VectorSubcoreMesh(core_axis_name="c", subcore_axis_name="s")
    f = pl.kernel(
        _gmf_body,
        out_type=jax.ShapeDtypeStruct((B,), jnp.float32),
        mesh=mesh,
        scratch_types=[
            pltpu.VMEM((PAD,), jnp.int32),
            pltpu.VMEM((PAD,), jnp.int32),
            pltpu.VMEM((HALF, D, 128), jnp.float32),
            pltpu.VMEM((HALF, D, 128), jnp.float32),
            pltpu.VMEM((2 * D, 16), jnp.float32),
            pltpu.VMEM((1, D), jnp.float32),
            pltpu.VMEM((16,), jnp.float32),
            pltpu.VMEM((PAD,), jnp.float32),
            pltpu.SemaphoreType.DMA,
            pltpu.SemaphoreType.DMA,
        ],
        compiler_params=pltpu.CompilerParams(needs_layout_passes=False),
    )
    return f(users, items, user_table.T, item_table.T, W, b)


def kernel(users, items, user_table, item_table, W, b):
    return _gmf(users, items, user_table, item_table, W, b)


# speedup vs baseline: 1.0155x; 1.0155x over previous
import functools

import jax
import jax.numpy as jnp
from jax import lax
from jax.experimental import pallas as pl
from jax.experimental.pallas import tpu as pltpu
from jax.experimental.pallas import tpu_sc as plsc

B = 16384
D = 32
NC = 2
NS = 16
NW = NC * NS
BPW = B // NW
GROUPS = BPW // 16


def _gmf_body(users_hbm, items_hbm, ut_hbm, it_hbm, w_hbm, b_hbm, out_hbm,
              uidx_v, iidx_v, win_v, uc_v, w_v, b_v, out_v,
              sem_u, sem_i):
    wid = lax.axis_index("s") * NC + lax.axis_index("c")
    base = wid * BPW

    pltpu.sync_copy(users_hbm.at[pl.ds(base, BPW)], uidx_v)
    pltpu.sync_copy(items_hbm.at[pl.ds(base, BPW)], iidx_v)
    pltpu.sync_copy(w_hbm, w_v)
    pltpu.sync_copy(b_hbm, b_v.at[pl.ds(0, 1)])

    wv0 = w_v[0, pl.ds(0, 16)]
    wv1 = w_v[0, pl.ds(16, 16)]
    bias = b_v[pl.ds(0, 16)][0]
    lane16 = lax.iota(jnp.int32, 16)

    def fire(tab_hbm, ivec, sem):
        copies = []
        for j in range(16):
            c = ivec[j]
            c128 = pl.multiple_of((c >> 7) << 7, 128)
            for i in range(4):
                copies.append(pltpu.async_copy(
                    tab_hbm.at[pl.ds(8 * i, 8), pl.ds(c128, 128)],
                    win_v.at[j, pl.ds(8 * i, 8), :], sem))
        return copies

    def group(g, carry):
        iv_u = uidx_v[pl.ds(g * 16, 16)]
        iv_i = iidx_v[pl.ds(g * 16, 16)]
        lv_u = iv_u & 127
        lv_i = iv_i & 127

        for cp in fire(ut_hbm, iv_u, sem_u):
            cp.wait()
        for d in range(D):
            dsplat = jnp.full((16,), d, jnp.int32)
            uc_v[d] = plsc.load_gather(win_v, [lane16, dsplat, lv_u])

        for cp in fire(it_hbm, iv_i, sem_i):
            cp.wait()
        acc = jnp.full((16,), bias, jnp.float32)
        for d in range(D):
            dsplat = jnp.full((16,), d, jnp.int32)
            i_d = plsc.load_gather(win_v, [lane16, dsplat, lv_i])
            wd = wv0[d] if d < 16 else wv1[d - 16]
            acc = acc + (uc_v[d] * i_d) * wd
        out_v[pl.ds(g * 16, 16)] = acc
        return carry

    lax.fori_loop(0, GROUPS, group, 0)

    pltpu.sync_copy(out_v, out_hbm.at[pl.ds(base, BPW)])


@jax.jit
def _gmf(users, items, user_table, item_table, W, b):
    mesh = plsc.VectorSubcoreMesh(core_axis_name="c", subcore_axis_name="s")
    f = pl.kernel(
        _gmf_body,
        out_type=jax.ShapeDtypeStruct((B,), jnp.float32),
        mesh=mesh,
        scratch_types=[
            pltpu.VMEM((BPW,), jnp.int32),
            pltpu.VMEM((BPW,), jnp.int32),
            pltpu.VMEM((16, D, 128), jnp.float32),
            pltpu.VMEM((D, 16), jnp.float32),
            pltpu.VMEM((1, D), jnp.float32),
            pltpu.VMEM((16,), jnp.float32),
            pltpu.VMEM((BPW,), jnp.float32),
            pltpu.SemaphoreType.DMA,
            pltpu.SemaphoreType.DMA,
        ],
        compiler_params=pltpu.CompilerParams(needs_layout_passes=False),
    )
    return f(users, items, user_table.T, item_table.T, W, b)


def kernel(users, items, user_table, item_table, W, b):
    return _gmf(users, items, user_table, item_table, W, b)


# speedup vs baseline: 1.0211x; 1.0055x over previous
import functools

import jax
import jax.numpy as jnp
from jax import lax
from jax.experimental import pallas as pl
from jax.experimental.pallas import tpu as pltpu
from jax.experimental.pallas import tpu_sc as plsc

B = 16384
D = 32
NC = 2
NS = 16
NW = NC * NS
BPW = B // NW
GROUPS = BPW // 16


def _gmf_body(users_hbm, items_hbm, ut_hbm, it_hbm, w_hbm, b_hbm, out_hbm,
              uidx_v, iidx_v, win_v, uc_v, w_v, b_v, out_v,
              sem_u, sem_i):
    wid = lax.axis_index("s") * NC + lax.axis_index("c")
    base = wid * BPW

    pltpu.sync_copy(users_hbm.at[pl.ds(base, BPW)], uidx_v)
    pltpu.sync_copy(items_hbm.at[pl.ds(base, BPW)], iidx_v)
    pltpu.sync_copy(w_hbm, w_v)
    pltpu.sync_copy(b_hbm, b_v.at[pl.ds(0, 1)])

    wv0 = w_v[0, pl.ds(0, 16)]
    wv1 = w_v[0, pl.ds(16, 16)]
    bias = b_v[pl.ds(0, 16)][0]
    lane16 = lax.iota(jnp.int32, 16)

    def fire(tab_hbm, ivec, sem):
        copies = []
        for j in range(16):
            c = ivec[j]
            c128 = pl.multiple_of((c >> 7) << 7, 128)
            copies.append(pltpu.async_copy(
                tab_hbm.at[:, pl.ds(c128, 128)], win_v.at[j], sem))
        return copies

    def group(g, carry):
        iv_u = uidx_v[pl.ds(g * 16, 16)]
        iv_i = iidx_v[pl.ds(g * 16, 16)]
        lv_u = iv_u & 127
        lv_i = iv_i & 127

        for cp in fire(ut_hbm, iv_u, sem_u):
            cp.wait()
        for d in range(D):
            dsplat = jnp.full((16,), d, jnp.int32)
            uc_v[d] = plsc.load_gather(win_v, [lane16, dsplat, lv_u])

        for cp in fire(it_hbm, iv_i, sem_i):
            cp.wait()
        acc = jnp.full((16,), bias, jnp.float32)
        for d in range(D):
            dsplat = jnp.full((16,), d, jnp.int32)
            i_d = plsc.load_gather(win_v, [lane16, dsplat, lv_i])
            wd = wv0[d] if d < 16 else wv1[d - 16]
            acc = acc + (uc_v[d] * i_d) * wd
        out_v[pl.ds(g * 16, 16)] = acc
        return carry

    lax.fori_loop(0, GROUPS, group, 0)

    pltpu.sync_copy(out_v, out_hbm.at[pl.ds(base, BPW)])


@jax.jit
def _gmf(users, items, user_table, item_table, W, b):
    mesh = plsc.VectorSubcoreMesh(core_axis_name="c", subcore_axis_name="s")
    f = pl.kernel(
        _gmf_body,
        out_type=jax.ShapeDtypeStruct((B,), jnp.float32),
        mesh=mesh,
        scratch_types=[
            pltpu.VMEM((BPW,), jnp.int32),
            pltpu.VMEM((BPW,), jnp.int32),
            pltpu.VMEM((16, D, 128), jnp.float32),
            pltpu.VMEM((D, 16), jnp.float32),
            pltpu.VMEM((1, D), jnp.float32),
            pltpu.VMEM((16,), jnp.float32),
            pltpu.VMEM((BPW,), jnp.float32),
            pltpu.SemaphoreType.DMA,
            pltpu.SemaphoreType.DMA,
        ],
        compiler_params=pltpu.CompilerParams(needs_layout_passes=False),
    )
    return f(users, items, user_table.T, item_table.T, W, b)


def kernel(users, items, user_table, item_table, W, b):
    return _gmf(users, items, user_table, item_table, W, b)
